# Initial kernel scaffold; baseline (speedup 1.0000x reference)
import jax
import jax.numpy as jnp
from jax.experimental import pallas as pl


def kernel(x, edge_index, edge_attr, batch, emb, fw1, fb1, fw2, fb2, lw, lb, ln_g, ln_b):
    raise NotImplementedError("write your pallas kernel here")


# speedup vs baseline: 2.7667x; 2.7667x over previous
import functools

import jax
import jax.numpy as jnp
import numpy as np
from jax import lax
from jax.experimental import pallas as pl
from jax.experimental.pallas import tpu as pltpu
from jax.experimental.pallas import tpu_sc as plsc

N = 10000
E = 320000
HID = 128
NUM_RBF = 50
RBF_PAD = 64
L = 4
G = 64
CUTOFF = 4.0
VOCAB = 128

_STEP = np.float32(CUTOFF) * (np.float32(1.0) / np.float32(NUM_RBF - 1))
_COEFF = -0.5 / float(_STEP) ** 2

_NBLK = 1000


def _embed_body(x_ref, emb_ref, o_ref):
    xv = x_ref[0]
    ids = lax.broadcasted_iota(jnp.int32, (VOCAB, _NBLK), 0)
    oh = (ids == jnp.broadcast_to(xv, (VOCAB, _NBLK))).astype(jnp.float32)
    o_ref[...] = lax.dot_general(oh, emb_ref[...], (((0,), (0,)), ((), ())),
                                 preferred_element_type=jnp.float32)


def _embed(x3, emb):
    return pl.pallas_call(
        _embed_body,
        grid=(N // _NBLK,),
        in_specs=[
            pl.BlockSpec((1, 1, _NBLK), lambda i: (i, 0, 0)),
            pl.BlockSpec((VOCAB, HID), lambda i: (0, 0)),
        ],
        out_specs=pl.BlockSpec((_NBLK, HID), lambda i: (i, 0)),
        out_shape=jax.ShapeDtypeStruct((N, HID), jnp.float32),
    )(x3, emb)


_EBLK = 2000


def _filter_body(attr_ref, fw1_ref, fb1_ref, fw2_ref, fb2_ref, o_ref):
    a = attr_ref[0]
    ab = jnp.broadcast_to(a, (RBF_PAD, _EBLK))
    off = lax.broadcasted_iota(jnp.int32, (RBF_PAD, _EBLK), 0).astype(jnp.float32) * _STEP
    d = ab - off
    rbf_t = jnp.exp(_COEFF * d * d)
    t = lax.dot_general(rbf_t, fw1_ref[...], (((0,), (0,)), ((), ())),
                        preferred_element_type=jnp.float32) + fb1_ref[...]
    t = jnp.maximum(t, 0.0)
    w = jnp.dot(t, fw2_ref[...], preferred_element_type=jnp.float32) + fb2_ref[...]
    o_ref[...] = w


def _filter(attr3, fw1p, fb1, fw2, fb2):
    return pl.pallas_call(
        _filter_body,
        grid=(E // _EBLK,),
        in_specs=[
            pl.BlockSpec((1, 1, _EBLK), lambda i: (i, 0, 0)),
            pl.BlockSpec((RBF_PAD, HID), lambda i: (0, 0)),
            pl.BlockSpec((1, HID), lambda i: (0, 0)),
            pl.BlockSpec((HID, HID), lambda i: (0, 0)),
            pl.BlockSpec((1, HID), lambda i: (0, 0)),
        ],
        out_specs=pl.BlockSpec((_EBLK, HID), lambda i: (i, 0)),
        out_shape=jax.ShapeDtypeStruct((E, HID), jnp.float32),
    )(attr3, fw1p, fb1, fw2, fb2)


def _xl_body(h_ref, lw_ref, lb_ref, o_ref):
    o_ref[...] = jnp.dot(h_ref[...], lw_ref[...],
                         preferred_element_type=jnp.float32) + lb_ref[...]


def _xl(h, lw, lb):
    return pl.pallas_call(
        _xl_body,
        grid=(N // _NBLK,),
        in_specs=[
            pl.BlockSpec((_NBLK, HID), lambda i: (i, 0)),
            pl.BlockSpec((HID, HID), lambda i: (0, 0)),
            pl.BlockSpec((1, HID), lambda i: (0, 0)),
        ],
        out_specs=pl.BlockSpec((_NBLK, HID), lambda i: (i, 0)),
        out_shape=jax.ShapeDtypeStruct((N, HID), jnp.float32),
    )(h, lw, lb)


def _combine_body(h_ref, p0_ref, p1_ref, g_ref, b_ref, o_ref):
    hn = h_ref[...] + p0_ref[...] + p1_ref[...]
    mu = jnp.mean(hn, axis=-1, keepdims=True)
    cen = hn - mu
    var = jnp.mean(cen * cen, axis=-1, keepdims=True)
    o_ref[...] = cen * lax.rsqrt(var + 1e-5) * g_ref[...] + b_ref[...]


def _combine(h, p0, p1, g, b):
    return pl.pallas_call(
        _combine_body,
        grid=(N // _NBLK,),
        in_specs=[
            pl.BlockSpec((_NBLK, HID), lambda i: (i, 0)),
            pl.BlockSpec((_NBLK, HID), lambda i: (i, 0)),
            pl.BlockSpec((_NBLK, HID), lambda i: (i, 0)),
            pl.BlockSpec((1, HID), lambda i: (0, 0)),
            pl.BlockSpec((1, HID), lambda i: (0, 0)),
        ],
        out_specs=pl.BlockSpec((_NBLK, HID), lambda i: (i, 0)),
        out_shape=jax.ShapeDtypeStruct((N, HID), jnp.float32),
    )(h, p0, p1, g, b)


def _pool_body(b3_ref, h_ref, o_ref, acc, cnt):
    i = pl.program_id(0)

    @pl.when(i == 0)
    def _():
        acc[...] = jnp.zeros((G, HID), jnp.float32)
        cnt[...] = jnp.zeros((G, HID), jnp.float32)

    bv = b3_ref[0]
    gids = lax.broadcasted_iota(jnp.int32, (G, _NBLK), 0)
    oh = (gids == jnp.broadcast_to(bv, (G, _NBLK))).astype(jnp.float32)
    acc[...] += jnp.dot(oh, h_ref[...], preferred_element_type=jnp.float32)
    cnt[...] += jnp.broadcast_to(jnp.sum(oh, axis=1, keepdims=True), (G, HID))

    @pl.when(i == pl.num_programs(0) - 1)
    def _():
        o_ref[...] = acc[...] / jnp.maximum(cnt[...], 1.0)


def _pool(b3, h):
    return pl.pallas_call(
        _pool_body,
        grid=(N // _NBLK,),
        in_specs=[
            pl.BlockSpec((1, 1, _NBLK), lambda i: (i, 0, 0)),
            pl.BlockSpec((_NBLK, HID), lambda i: (i, 0)),
        ],
        out_specs=pl.BlockSpec((G, HID), lambda i: (0, 0)),
        out_shape=jax.ShapeDtypeStruct((G, HID), jnp.float32),
        scratch_shapes=[
            pltpu.VMEM((G, HID), jnp.float32),
            pltpu.VMEM((G, HID), jnp.float32),
        ],
    )(b3, h)


_NCORE = 2
_NSUB = 16
_EC = 80
_E_PER_TILE = E // (_NCORE * _NSUB)
_NCHUNK = _E_PER_TILE // _EC
_NPAD = 10240
_R_PER_TILE = _NPAD // _NSUB
_ZR = 128


def _sc_body(xl_hbm, w_hbm, col_hbm, row_hbm, out_hbm,
             acc, col_v, row_v, w_v, g_v, zbuf, sem):
    cid = lax.axis_index("c")
    sid = lax.axis_index("s")

    def _zrow(i, carry):
        for j in range(HID // 16):
            zbuf[i, pl.ds(j * 16, 16)] = jnp.zeros((16,), jnp.float32)
        return carry
    lax.fori_loop(0, _ZR, _zrow, 0)
    for t in range(_R_PER_TILE // _ZR):
        pltpu.sync_copy(zbuf, acc.at[pl.ds(sid * _R_PER_TILE + t * _ZR, _ZR)])
    plsc.subcore_barrier()

    base = cid * (E // _NCORE) + sid * _E_PER_TILE

    def _chunk(k, carry):
        off = base + k * _EC
        pltpu.sync_copy(col_hbm.at[pl.ds(off, _EC)], col_v)
        pltpu.sync_copy(row_hbm.at[pl.ds(off, _EC)], row_v)
        pltpu.sync_copy(w_hbm.at[pl.ds(off, _EC)], w_v)
        pltpu.async_copy(xl_hbm.at[col_v], g_v, sem).wait()

        def _mul(e, c2):
            for j in range(HID // 16):
                s = pl.ds(j * 16, 16)
                g_v[e, s] = g_v[e, s] * w_v[e, s]
            return c2
        lax.fori_loop(0, _EC, _mul, 0)
        pltpu.sync_copy(g_v, acc.at[row_v], add=True)
        return carry

    lax.fori_loop(0, _NCHUNK, _chunk, 0)
    plsc.subcore_barrier()

    for t in range(_R_PER_TILE // _ZR):
        r0 = sid * _R_PER_TILE + t * _ZR
        pltpu.sync_copy(acc.at[pl.ds(r0, _ZR)], out_hbm.at[cid, pl.ds(r0, _ZR)])


@functools.partial(jax.jit, static_argnames=())
def _sc_scatter(xl, w, col, row):
    mesh = plsc.VectorSubcoreMesh(core_axis_name="c", subcore_axis_name="s")
    kfn = pl.kernel(
        _sc_body,
        mesh=mesh,
        out_type=jax.ShapeDtypeStruct((_NCORE, _NPAD, HID), jnp.float32),
        scratch_types=[
            pltpu.VMEM_SHARED((_NPAD, HID), jnp.float32),
            pltpu.VMEM((_EC,), jnp.int32),
            pltpu.VMEM((_EC,), jnp.int32),
            pltpu.VMEM((_EC, HID), jnp.float32),
            pltpu.VMEM((_EC, HID), jnp.float32),
            pltpu.VMEM((_ZR, HID), jnp.float32),
            pltpu.SemaphoreType.DMA,
        ],
    )
    return kfn(xl, w, col, row)


def kernel(x, edge_index, edge_attr, batch, emb, fw1, fb1, fw2, fb2, lw, lb, ln_g, ln_b):
    x3 = x.astype(jnp.int32).reshape(N // _NBLK, 1, _NBLK)
    b3 = batch.astype(jnp.int32).reshape(N // _NBLK, 1, _NBLK)
    attr3 = edge_attr.reshape(E // _EBLK, 1, _EBLK)
    row = edge_index[0].astype(jnp.int32)
    col = edge_index[1].astype(jnp.int32)
    fw1p = jnp.pad(fw1, ((0, 0), (0, RBF_PAD - NUM_RBF), (0, 0)))

    h = _embed(x3, emb)
    for i in range(L):
        w = _filter(attr3, fw1p[i], fb1[i][None, :], fw2[i], fb2[i][None, :])
        xl = _xl(h, lw[i], lb[i][None, :])
        parts = _sc_scatter(xl, w, col, row)
        h = _combine(h, parts[0, :N], parts[1, :N], ln_g[i][None, :], ln_b[i][None, :])
    return _pool(b3, h)


# speedup vs baseline: 4.5346x; 1.6390x over previous
import functools

import jax
import jax.numpy as jnp
import numpy as np
from jax import lax
from jax.experimental import pallas as pl
from jax.experimental.pallas import tpu as pltpu
from jax.experimental.pallas import tpu_sc as plsc

N = 10000
E = 320000
HID = 128
NUM_RBF = 50
RBF_PAD = 64
L = 4
G = 64
CUTOFF = 4.0
VOCAB = 128

_STEP = np.float32(CUTOFF) * (np.float32(1.0) / np.float32(NUM_RBF - 1))
_COEFF = -0.5 / float(_STEP) ** 2

_NBLK = 1000


def _embed_body(x_ref, emb_ref, o_ref):
    xv = x_ref[0]
    ids = lax.broadcasted_iota(jnp.int32, (VOCAB, _NBLK), 0)
    oh = (ids == jnp.broadcast_to(xv, (VOCAB, _NBLK))).astype(jnp.float32)
    o_ref[...] = lax.dot_general(oh, emb_ref[...], (((0,), (0,)), ((), ())),
                                 preferred_element_type=jnp.float32)


def _embed(x3, emb):
    return pl.pallas_call(
        _embed_body,
        grid=(N // _NBLK,),
        in_specs=[
            pl.BlockSpec((1, 1, _NBLK), lambda i: (i, 0, 0)),
            pl.BlockSpec((VOCAB, HID), lambda i: (0, 0)),
        ],
        out_specs=pl.BlockSpec((_NBLK, HID), lambda i: (i, 0)),
        out_shape=jax.ShapeDtypeStruct((N, HID), jnp.float32),
    )(x3, emb)


_EBLK = 2000


def _filter_body(attr_ref, fw1_ref, fb1_ref, fw2_ref, fb2_ref, o_ref):
    a = attr_ref[0]
    ab = jnp.broadcast_to(a, (RBF_PAD, _EBLK))
    off = lax.broadcasted_iota(jnp.int32, (RBF_PAD, _EBLK), 0).astype(jnp.float32) * _STEP
    d = ab - off
    rbf_t = jnp.exp(_COEFF * d * d)
    t = lax.dot_general(rbf_t, fw1_ref[...], (((0,), (0,)), ((), ())),
                        preferred_element_type=jnp.float32) + fb1_ref[...]
    t = jnp.maximum(t, 0.0)
    w = jnp.dot(t, fw2_ref[...], preferred_element_type=jnp.float32) + fb2_ref[...]
    o_ref[...] = w


def _filter(attr3, fw1p, fb1, fw2, fb2):
    return pl.pallas_call(
        _filter_body,
        grid=(E // _EBLK,),
        in_specs=[
            pl.BlockSpec((1, 1, _EBLK), lambda i: (i, 0, 0)),
            pl.BlockSpec((RBF_PAD, HID), lambda i: (0, 0)),
            pl.BlockSpec((1, HID), lambda i: (0, 0)),
            pl.BlockSpec((HID, HID), lambda i: (0, 0)),
            pl.BlockSpec((1, HID), lambda i: (0, 0)),
        ],
        out_specs=pl.BlockSpec((_EBLK, HID), lambda i: (i, 0)),
        out_shape=jax.ShapeDtypeStruct((E, HID), jnp.float32),
    )(attr3, fw1p, fb1, fw2, fb2)


def _xl_body(h_ref, lw_ref, lb_ref, o_ref):
    o_ref[...] = jnp.dot(h_ref[...], lw_ref[...],
                         preferred_element_type=jnp.float32) + lb_ref[...]


def _xl(h, lw, lb):
    return pl.pallas_call(
        _xl_body,
        grid=(N // _NBLK,),
        in_specs=[
            pl.BlockSpec((_NBLK, HID), lambda i: (i, 0)),
            pl.BlockSpec((HID, HID), lambda i: (0, 0)),
            pl.BlockSpec((1, HID), lambda i: (0, 0)),
        ],
        out_specs=pl.BlockSpec((_NBLK, HID), lambda i: (i, 0)),
        out_shape=jax.ShapeDtypeStruct((N, HID), jnp.float32),
    )(h, lw, lb)


def _combine_body(h_ref, p0_ref, p1_ref, g_ref, b_ref, o_ref):
    hn = h_ref[...] + p0_ref[...] + p1_ref[...]
    mu = jnp.mean(hn, axis=-1, keepdims=True)
    cen = hn - mu
    var = jnp.mean(cen * cen, axis=-1, keepdims=True)
    o_ref[...] = cen * lax.rsqrt(var + 1e-5) * g_ref[...] + b_ref[...]


def _combine(h, p0, p1, g, b):
    return pl.pallas_call(
        _combine_body,
        grid=(N // _NBLK,),
        in_specs=[
            pl.BlockSpec((_NBLK, HID), lambda i: (i, 0)),
            pl.BlockSpec((_NBLK, HID), lambda i: (i, 0)),
            pl.BlockSpec((_NBLK, HID), lambda i: (i, 0)),
            pl.BlockSpec((1, HID), lambda i: (0, 0)),
            pl.BlockSpec((1, HID), lambda i: (0, 0)),
        ],
        out_specs=pl.BlockSpec((_NBLK, HID), lambda i: (i, 0)),
        out_shape=jax.ShapeDtypeStruct((N, HID), jnp.float32),
    )(h, p0, p1, g, b)


def _pool_body(b3_ref, h_ref, o_ref, acc, cnt):
    i = pl.program_id(0)

    @pl.when(i == 0)
    def _():
        acc[...] = jnp.zeros((G, HID), jnp.float32)
        cnt[...] = jnp.zeros((G, HID), jnp.float32)

    bv = b3_ref[0]
    gids = lax.broadcasted_iota(jnp.int32, (G, _NBLK), 0)
    oh = (gids == jnp.broadcast_to(bv, (G, _NBLK))).astype(jnp.float32)
    acc[...] += jnp.dot(oh, h_ref[...], preferred_element_type=jnp.float32)
    cnt[...] += jnp.broadcast_to(jnp.sum(oh, axis=1, keepdims=True), (G, HID))

    @pl.when(i == pl.num_programs(0) - 1)
    def _():
        o_ref[...] = acc[...] / jnp.maximum(cnt[...], 1.0)


def _pool(b3, h):
    return pl.pallas_call(
        _pool_body,
        grid=(N // _NBLK,),
        in_specs=[
            pl.BlockSpec((1, 1, _NBLK), lambda i: (i, 0, 0)),
            pl.BlockSpec((_NBLK, HID), lambda i: (i, 0)),
        ],
        out_specs=pl.BlockSpec((G, HID), lambda i: (0, 0)),
        out_shape=jax.ShapeDtypeStruct((G, HID), jnp.float32),
        scratch_shapes=[
            pltpu.VMEM((G, HID), jnp.float32),
            pltpu.VMEM((G, HID), jnp.float32),
        ],
    )(b3, h)


_NCORE = 2
_NSUB = 16
_EC = 80
_E_PER_TILE = E // (_NCORE * _NSUB)
_NCHUNK = _E_PER_TILE // _EC
_NPAD = 10240
_R_PER_TILE = _NPAD // _NSUB
_ZR = 128


_NBUF = 2


def _sc_body(xl_hbm, w_hbm, col_hbm, row_hbm, out_hbm, acc, *scr):
    col_v = scr[0:_NBUF]
    row_v = scr[_NBUF:2 * _NBUF]
    w_v = scr[2 * _NBUF:3 * _NBUF]
    g_v = scr[3 * _NBUF:4 * _NBUF]
    sem_i = scr[4 * _NBUF:5 * _NBUF]
    sem_w = scr[5 * _NBUF:6 * _NBUF]
    sem_g = scr[6 * _NBUF:7 * _NBUF]

    cid = lax.axis_index("c")
    sid = lax.axis_index("s")

    def _zrow(i, carry):
        for j in range(HID // 16):
            g_v[0][i, pl.ds(j * 16, 16)] = jnp.zeros((16,), jnp.float32)
        return carry
    lax.fori_loop(0, _EC, _zrow, 0)
    for t in range(_R_PER_TILE // _EC):
        pltpu.sync_copy(g_v[0], acc.at[pl.ds(sid * _R_PER_TILE + t * _EC, _EC)])
    plsc.subcore_barrier()

    base = cid * (E // _NCORE) + sid * _E_PER_TILE

    def _do_chunks(k0, nchunks):
        d_col, d_row, d_w = [], [], []
        for b in range(nchunks):
            off = base + (k0 + b) * _EC
            d_col.append(pltpu.async_copy(col_hbm.at[pl.ds(off, _EC)], col_v[b], sem_i[b]))
            d_row.append(pltpu.async_copy(row_hbm.at[pl.ds(off, _EC)], row_v[b], sem_i[b]))
            d_w.append(pltpu.async_copy(w_hbm.at[pl.ds(off, _EC)], w_v[b], sem_w[b]))
        d_g = []
        for b in range(nchunks):
            d_col[b].wait()
            d_row[b].wait()
            d_g.append(pltpu.async_copy(xl_hbm.at[col_v[b]], g_v[b], sem_g[b]))
        for b in range(nchunks):
            d_g[b].wait()
            d_w[b].wait()

            def _mul(e, c2, _b=b):
                for j in range(HID // 16):
                    s = pl.ds(j * 16, 16)
                    g_v[_b][e, s] = g_v[_b][e, s] * w_v[_b][e, s]
                return c2
            lax.fori_loop(0, _EC, _mul, 0)
            pltpu.sync_copy(g_v[b], acc.at[row_v[b]], add=True)

    def _group(k, carry):
        _do_chunks(k * _NBUF, _NBUF)
        return carry

    lax.fori_loop(0, _NCHUNK // _NBUF, _group, 0)
    if _NCHUNK % _NBUF:
        _do_chunks(_NCHUNK - _NCHUNK % _NBUF, _NCHUNK % _NBUF)
    plsc.subcore_barrier()

    for t in range(_R_PER_TILE // _ZR):
        r0 = sid * _R_PER_TILE + t * _ZR
        pltpu.sync_copy(acc.at[pl.ds(r0, _ZR)], out_hbm.at[cid, pl.ds(r0, _ZR)])


@functools.partial(jax.jit, static_argnames=())
def _sc_scatter(xl, w, col, row):
    mesh = plsc.VectorSubcoreMesh(core_axis_name="c", subcore_axis_name="s")
    kfn = pl.kernel(
        _sc_body,
        mesh=mesh,
        out_type=jax.ShapeDtypeStruct((_NCORE, _NPAD, HID), jnp.float32),
        scratch_types=(
            [pltpu.VMEM_SHARED((_NPAD, HID), jnp.float32)]
            + [pltpu.VMEM((_EC,), jnp.int32) for _ in range(2 * _NBUF)]
            + [pltpu.VMEM((_EC, HID), jnp.float32) for _ in range(2 * _NBUF)]
            + [pltpu.SemaphoreType.DMA for _ in range(3 * _NBUF)]
        ),
    )
    return kfn(xl, w, col, row)


def kernel(x, edge_index, edge_attr, batch, emb, fw1, fb1, fw2, fb2, lw, lb, ln_g, ln_b):
    x3 = x.astype(jnp.int32).reshape(N // _NBLK, 1, _NBLK)
    b3 = batch.astype(jnp.int32).reshape(N // _NBLK, 1, _NBLK)
    attr3 = edge_attr.reshape(E // _EBLK, 1, _EBLK)
    row = edge_index[0].astype(jnp.int32)
    col = edge_index[1].astype(jnp.int32)
    fw1p = jnp.pad(fw1, ((0, 0), (0, RBF_PAD - NUM_RBF), (0, 0)))

    h = _embed(x3, emb)
    for i in range(L):
        w = _filter(attr3, fw1p[i], fb1[i][None, :], fw2[i], fb2[i][None, :])
        xl = _xl(h, lw[i], lb[i][None, :])
        parts = _sc_scatter(xl, w, col, row)
        h = _combine(h, parts[0, :N], parts[1, :N], ln_g[i][None, :], ln_b[i][None, :])
    return _pool(b3, h)


# speedup vs baseline: 6.4131x; 1.4143x over previous
import functools

import jax
import jax.numpy as jnp
import numpy as np
from jax import lax
from jax.experimental import pallas as pl
from jax.experimental.pallas import tpu as pltpu
from jax.experimental.pallas import tpu_sc as plsc

N = 10000
E = 320000
HID = 128
NUM_RBF = 50
RBF_PAD = 64
L = 4
G = 64
CUTOFF = 4.0
VOCAB = 128

_STEP = np.float32(CUTOFF) * (np.float32(1.0) / np.float32(NUM_RBF - 1))
_COEFF = -0.5 / float(_STEP) ** 2

_NBLK = 1000


def _embed_body(x_ref, emb_ref, o_ref):
    xv = x_ref[0]
    ids = lax.broadcasted_iota(jnp.int32, (VOCAB, _NBLK), 0)
    oh = (ids == jnp.broadcast_to(xv, (VOCAB, _NBLK))).astype(jnp.float32)
    o_ref[...] = lax.dot_general(oh, emb_ref[...], (((0,), (0,)), ((), ())),
                                 preferred_element_type=jnp.float32)


def _embed(x3, emb):
    return pl.pallas_call(
        _embed_body,
        grid=(N // _NBLK,),
        in_specs=[
            pl.BlockSpec((1, 1, _NBLK), lambda i: (i, 0, 0)),
            pl.BlockSpec((VOCAB, HID), lambda i: (0, 0)),
        ],
        out_specs=pl.BlockSpec((_NBLK, HID), lambda i: (i, 0)),
        out_shape=jax.ShapeDtypeStruct((N, HID), jnp.float32),
    )(x3, emb)


_EBLK = 2000


def _filter_body(attr_ref, fw1_ref, fb1_ref, fw2_ref, fb2_ref, o_ref):
    a = attr_ref[0]
    ab = jnp.broadcast_to(a, (RBF_PAD, _EBLK))
    off = lax.broadcasted_iota(jnp.int32, (RBF_PAD, _EBLK), 0).astype(jnp.float32) * _STEP
    d = ab - off
    rbf_t = jnp.exp(_COEFF * d * d)
    t = lax.dot_general(rbf_t, fw1_ref[...], (((0,), (0,)), ((), ())),
                        preferred_element_type=jnp.float32) + fb1_ref[...]
    t = jnp.maximum(t, 0.0)
    w = jnp.dot(t, fw2_ref[...], preferred_element_type=jnp.float32) + fb2_ref[...]
    o_ref[...] = w


def _filter(attr3, fw1p, fb1, fw2, fb2):
    return pl.pallas_call(
        _filter_body,
        grid=(E // _EBLK,),
        in_specs=[
            pl.BlockSpec((1, 1, _EBLK), lambda i: (i, 0, 0)),
            pl.BlockSpec((RBF_PAD, HID), lambda i: (0, 0)),
            pl.BlockSpec((1, HID), lambda i: (0, 0)),
            pl.BlockSpec((HID, HID), lambda i: (0, 0)),
            pl.BlockSpec((1, HID), lambda i: (0, 0)),
        ],
        out_specs=pl.BlockSpec((_EBLK, HID), lambda i: (i, 0)),
        out_shape=jax.ShapeDtypeStruct((E, HID), jnp.float32),
    )(attr3, fw1p, fb1, fw2, fb2)


def _xl_body(h_ref, lw_ref, lb_ref, o_ref):
    o_ref[...] = jnp.dot(h_ref[...], lw_ref[...],
                         preferred_element_type=jnp.float32) + lb_ref[...]


def _xl(h, lw, lb):
    return pl.pallas_call(
        _xl_body,
        grid=(N // _NBLK,),
        in_specs=[
            pl.BlockSpec((_NBLK, HID), lambda i: (i, 0)),
            pl.BlockSpec((HID, HID), lambda i: (0, 0)),
            pl.BlockSpec((1, HID), lambda i: (0, 0)),
        ],
        out_specs=pl.BlockSpec((_NBLK, HID), lambda i: (i, 0)),
        out_shape=jax.ShapeDtypeStruct((N, HID), jnp.float32),
    )(h, lw, lb)


def _combine_body(h_ref, p0_ref, p1_ref, g_ref, b_ref, o_ref):
    hn = h_ref[...] + p0_ref[...] + p1_ref[...]
    mu = jnp.mean(hn, axis=-1, keepdims=True)
    cen = hn - mu
    var = jnp.mean(cen * cen, axis=-1, keepdims=True)
    o_ref[...] = cen * lax.rsqrt(var + 1e-5) * g_ref[...] + b_ref[...]


def _combine(h, p0, p1, g, b):
    return pl.pallas_call(
        _combine_body,
        grid=(N // _NBLK,),
        in_specs=[
            pl.BlockSpec((_NBLK, HID), lambda i: (i, 0)),
            pl.BlockSpec((_NBLK, HID), lambda i: (i, 0)),
            pl.BlockSpec((_NBLK, HID), lambda i: (i, 0)),
            pl.BlockSpec((1, HID), lambda i: (0, 0)),
            pl.BlockSpec((1, HID), lambda i: (0, 0)),
        ],
        out_specs=pl.BlockSpec((_NBLK, HID), lambda i: (i, 0)),
        out_shape=jax.ShapeDtypeStruct((N, HID), jnp.float32),
    )(h, p0, p1, g, b)


def _pool_body(b3_ref, h_ref, o_ref, acc, cnt):
    i = pl.program_id(0)

    @pl.when(i == 0)
    def _():
        acc[...] = jnp.zeros((G, HID), jnp.float32)
        cnt[...] = jnp.zeros((G, HID), jnp.float32)

    bv = b3_ref[0]
    gids = lax.broadcasted_iota(jnp.int32, (G, _NBLK), 0)
    oh = (gids == jnp.broadcast_to(bv, (G, _NBLK))).astype(jnp.float32)
    acc[...] += jnp.dot(oh, h_ref[...], preferred_element_type=jnp.float32)
    cnt[...] += jnp.broadcast_to(jnp.sum(oh, axis=1, keepdims=True), (G, HID))

    @pl.when(i == pl.num_programs(0) - 1)
    def _():
        o_ref[...] = acc[...] / jnp.maximum(cnt[...], 1.0)


def _pool(b3, h):
    return pl.pallas_call(
        _pool_body,
        grid=(N // _NBLK,),
        in_specs=[
            pl.BlockSpec((1, 1, _NBLK), lambda i: (i, 0, 0)),
            pl.BlockSpec((_NBLK, HID), lambda i: (i, 0)),
        ],
        out_specs=pl.BlockSpec((G, HID), lambda i: (0, 0)),
        out_shape=jax.ShapeDtypeStruct((G, HID), jnp.float32),
        scratch_shapes=[
            pltpu.VMEM((G, HID), jnp.float32),
            pltpu.VMEM((G, HID), jnp.float32),
        ],
    )(b3, h)


_NCORE = 2
_NSUB = 16
_EC = 80
_E_PER_TILE = E // (_NCORE * _NSUB)
_NCHUNK = _E_PER_TILE // _EC
_NPAD = 10240
_R_PER_TILE = _NPAD // _NSUB
_ZR = 128


_NBUF = 2


def _sc_body(xl_hbm, w_hbm, col_hbm, row_hbm, out_hbm, acc, *scr):
    col_v = scr[0:_NBUF]
    row_v = scr[_NBUF:2 * _NBUF]
    w_v = scr[2 * _NBUF:3 * _NBUF]
    g_v = scr[3 * _NBUF:4 * _NBUF]
    sem_i = scr[4 * _NBUF:5 * _NBUF]
    sem_w = scr[5 * _NBUF:6 * _NBUF]
    sem_g = scr[6 * _NBUF:7 * _NBUF]

    cid = lax.axis_index("c")
    sid = lax.axis_index("s")

    def _zrow(i, carry):
        for j in range(HID // 16):
            g_v[0][i, pl.ds(j * 16, 16)] = jnp.zeros((16,), jnp.float32)
        return carry
    lax.fori_loop(0, _EC, _zrow, 0)
    for t in range(_R_PER_TILE // _EC):
        pltpu.sync_copy(g_v[0], acc.at[pl.ds(sid * _R_PER_TILE + t * _EC, _EC)])
    plsc.subcore_barrier()

    base = cid * (E // _NCORE) + sid * _E_PER_TILE

    def _issue_loads(c, b):
        off = base + c * _EC
        pltpu.async_copy(col_hbm.at[pl.ds(off, _EC)], col_v[b], sem_i[b])
        pltpu.async_copy(row_hbm.at[pl.ds(off, _EC)], row_v[b], sem_i[b])
        pltpu.async_copy(w_hbm.at[pl.ds(off, _EC)], w_v[b], sem_w[b])

    def _wait_idx(b):
        pltpu.make_async_copy(col_hbm.at[pl.ds(0, _EC)], col_v[b], sem_i[b]).wait()
        pltpu.make_async_copy(row_hbm.at[pl.ds(0, _EC)], row_v[b], sem_i[b]).wait()

    def _wait_w(b):
        pltpu.make_async_copy(w_hbm.at[pl.ds(0, _EC)], w_v[b], sem_w[b]).wait()

    def _issue_gather(b):
        pltpu.async_copy(xl_hbm.at[col_v[b]], g_v[b], sem_g[b])

    def _wait_gather(b):
        pltpu.make_async_copy(xl_hbm.at[col_v[b]], g_v[b], sem_g[b]).wait()

    def _mul_scatter(b):
        _wait_gather(b)
        _wait_w(b)

        @functools.partial(plsc.parallel_loop, 0, _EC, unroll=4)
        def _m(e):
            for j in range(HID // 16):
                s = pl.ds(j * 16, 16)
                g_v[b][e, s] = g_v[b][e, s] * w_v[b][e, s]
        pltpu.sync_copy(g_v[b], acc.at[row_v[b]], add=True)

    _issue_loads(0, 0)
    _issue_loads(1, 1)
    _wait_idx(0)
    _issue_gather(0)

    def _iter(k, carry):
        c2 = lax.rem(2 * k + 2, _NCHUNK)
        c3 = lax.rem(2 * k + 3, _NCHUNK)
        _wait_idx(1)
        _issue_gather(1)
        _mul_scatter(0)
        _issue_loads(c2, 0)
        _mul_scatter(1)
        _issue_loads(c3, 1)
        _wait_idx(0)
        _issue_gather(0)
        return carry

    lax.fori_loop(0, _NCHUNK // 2, _iter, 0)
    _mul_scatter(0)
    _wait_idx(1)
    _wait_w(1)
    plsc.subcore_barrier()

    for t in range(_R_PER_TILE // _ZR):
        r0 = sid * _R_PER_TILE + t * _ZR
        pltpu.sync_copy(acc.at[pl.ds(r0, _ZR)], out_hbm.at[cid, pl.ds(r0, _ZR)])


@functools.partial(jax.jit, static_argnames=())
def _sc_scatter(xl, w, col, row):
    mesh = plsc.VectorSubcoreMesh(core_axis_name="c", subcore_axis_name="s")
    kfn = pl.kernel(
        _sc_body,
        mesh=mesh,
        out_type=jax.ShapeDtypeStruct((_NCORE, _NPAD, HID), jnp.float32),
        scratch_types=(
            [pltpu.VMEM_SHARED((_NPAD, HID), jnp.float32)]
            + [pltpu.VMEM((_EC,), jnp.int32) for _ in range(2 * _NBUF)]
            + [pltpu.VMEM((_EC, HID), jnp.float32) for _ in range(2 * _NBUF)]
            + [pltpu.SemaphoreType.DMA for _ in range(3 * _NBUF)]
        ),
    )
    return kfn(xl, w, col, row)


def kernel(x, edge_index, edge_attr, batch, emb, fw1, fb1, fw2, fb2, lw, lb, ln_g, ln_b):
    x3 = x.astype(jnp.int32).reshape(N // _NBLK, 1, _NBLK)
    b3 = batch.astype(jnp.int32).reshape(N // _NBLK, 1, _NBLK)
    attr3 = edge_attr.reshape(E // _EBLK, 1, _EBLK)
    row = edge_index[0].astype(jnp.int32)
    col = edge_index[1].astype(jnp.int32)
    fw1p = jnp.pad(fw1, ((0, 0), (0, RBF_PAD - NUM_RBF), (0, 0)))

    h = _embed(x3, emb)
    for i in range(L):
        w = _filter(attr3, fw1p[i], fb1[i][None, :], fw2[i], fb2[i][None, :])
        xl = _xl(h, lw[i], lb[i][None, :])
        parts = _sc_scatter(xl, w, col, row)
        h = _combine(h, parts[0, :N], parts[1, :N], ln_g[i][None, :], ln_b[i][None, :])
    return _pool(b3, h)
